# baseline (speedup 1.0000x reference)
import functools

import jax
import jax.numpy as jnp
from jax import lax
from jax.experimental import pallas as pl
from jax.experimental.pallas import tpu as pltpu
from jax.experimental.pallas import tpu_sc as plsc

SIGMA2 = 1.0
EPSILON = 0.25
D2_EPS = 1e-2

PAD = 16
SCW = 128
TI = 512
TJ = 512
NSUB = 16


def _make_scatter_sc(n_rows, m_rows):
    rows_per_w = n_rows // NSUB
    idx_per_w = m_rows // NSUB
    mesh = plsc.VectorSubcoreMesh(core_axis_name="c", subcore_axis_name="s")

    @functools.partial(
        pl.kernel,
        out_type=jax.ShapeDtypeStruct((n_rows, SCW), jnp.float32),
        mesh=mesh,
        scratch_types=[
            pltpu.VMEM((idx_per_w,), jnp.int32),
            pltpu.VMEM((idx_per_w, SCW), jnp.float32),
            pltpu.SemaphoreType.DMA,
        ],
    )
    def scatter(fg_hbm, bg_hbm, idx_hbm, out_hbm, idx_v, rows_v, sem):
        cid = lax.axis_index("c")
        sid = lax.axis_index("s")

        @pl.when(cid == 0)
        def _copy_bg():
            base = sid * rows_per_w
            pltpu.sync_copy(
                bg_hbm.at[pl.ds(base, rows_per_w)],
                out_hbm.at[pl.ds(base, rows_per_w)],
            )

        plsc.subcore_barrier()

        @pl.when(cid == 0)
        def _scatter_fg():
            ibase = sid * idx_per_w
            pltpu.sync_copy(idx_hbm.at[pl.ds(ibase, idx_per_w)], idx_v)
            pltpu.sync_copy(fg_hbm.at[pl.ds(ibase, idx_per_w)], rows_v)
            pltpu.async_copy(rows_v, out_hbm.at[idx_v], sem).wait()

    return scatter


def _energy_kernel(a_ref, bt_ref, out_ref):
    i = pl.program_id(0)
    j = pl.program_id(1)
    a = a_ref[...]
    bt = bt_ref[...]
    g = lax.dot_general(a, bt, (((1,), (0,)), ((), ())),
                        preferred_element_type=jnp.float32)
    sqa = jnp.sum(a * a, axis=1, keepdims=True)
    sqb = jnp.sum(bt * bt, axis=0, keepdims=True)
    d2 = jnp.maximum(sqa + sqb - 2.0 * g, 0.0) + D2_EPS
    r = SIGMA2 / d2
    inv6 = r * r * r
    e = (4.0 * EPSILON) * (inv6 * inv6 - inv6)

    def diag_sum():
        rows = lax.broadcasted_iota(jnp.int32, (TI, TJ), 0)
        cols = lax.broadcasted_iota(jnp.int32, (TI, TJ), 1)
        return jnp.sum(jnp.where(rows == cols, 0.0, e))

    part = lax.cond(i == j, diag_sum, lambda: jnp.sum(e))

    @pl.when((i == 0) & (j == 0))
    def _init():
        out_ref[...] = jnp.zeros((1, 1), jnp.float32)

    out_ref[...] += jnp.reshape(0.5 * part, (1, 1))


def _energy_call(full_p, full_t):
    n_rows = full_p.shape[0]
    nbi = n_rows // TI
    nbj = n_rows // TJ
    out = pl.pallas_call(
        _energy_kernel,
        grid=(nbi, nbj),
        in_specs=[
            pl.BlockSpec((TI, PAD), lambda i, j: (i, 0)),
            pl.BlockSpec((PAD, TJ), lambda i, j: (0, j)),
        ],
        out_specs=pl.BlockSpec((1, 1), lambda i, j: (0, 0)),
        out_shape=jax.ShapeDtypeStruct((1, 1), jnp.float32),
    )(full_p, full_t)
    return out[0, 0]


def kernel(fg, bg, mask_idx):
    m_rows = fg.shape[0]
    n_rows = bg.shape[0]
    fg_p = jnp.pad(fg.astype(jnp.float32), ((0, 0), (0, SCW - fg.shape[1])))
    bg_p = jnp.pad(bg.astype(jnp.float32), ((0, 0), (0, SCW - bg.shape[1])))
    idx = mask_idx.astype(jnp.int32)
    full_w = _make_scatter_sc(n_rows, m_rows)(fg_p, bg_p, idx)
    full_p = full_w[:, :PAD]
    return _energy_call(full_p, full_p.T)


# speedup vs baseline: 2.0186x; 2.0186x over previous
import functools

import jax
import jax.numpy as jnp
from jax import lax
from jax.experimental import pallas as pl
from jax.experimental.pallas import tpu as pltpu
from jax.experimental.pallas import tpu_sc as plsc

SIGMA2 = 1.0
EPSILON = 0.25
D2_EPS = 1e-2

PAD = 16
LANES = 16
NW = 32


def _make_scatter_sc(n_rows, m_rows):
    rows_per_w = n_rows // NW
    n_chunks = m_rows // LANES
    mesh = plsc.VectorSubcoreMesh(core_axis_name="c", subcore_axis_name="s")

    @functools.partial(
        pl.kernel,
        out_type=jax.ShapeDtypeStruct((n_rows * PAD,), jnp.float32),
        mesh=mesh,
        scratch_types=[
            pltpu.VMEM((rows_per_w * PAD,), jnp.float32),
            pltpu.VMEM((m_rows,), jnp.int32),
            pltpu.VMEM((3 * m_rows,), jnp.float32),
        ],
        compiler_params=pltpu.CompilerParams(needs_layout_passes=False),
    )
    def scatter(fgt_hbm, bg_hbm, idx_hbm, out_hbm, out_v, idx_v, fgt_v):
        cid = lax.axis_index("c")
        sid = lax.axis_index("s")
        wid = sid * 2 + cid
        base = wid * rows_per_w

        pltpu.sync_copy(bg_hbm.at[pl.ds(base * PAD, rows_per_w * PAD)], out_v)
        pltpu.sync_copy(idx_hbm, idx_v)
        pltpu.sync_copy(fgt_hbm, fgt_v)

        def body(k, carry):
            tgt = idx_v[pl.ds(k * LANES, LANES)]
            ok = (tgt >= base) & (tgt < base + rows_per_w)
            local = jnp.where(ok, tgt - base, 0)
            for c in range(3):
                vals = fgt_v[pl.ds(c * m_rows + k * LANES, LANES)]
                plsc.store_scatter(out_v, [local * PAD + c], vals, mask=ok)
            return carry

        lax.fori_loop(0, n_chunks, body, 0)
        pltpu.sync_copy(out_v, out_hbm.at[pl.ds(base * PAD, rows_per_w * PAD)])

    return scatter


TI = 512
TJ = 512


def _energy_kernel(a_ref, bt_ref, out_ref):
    i = pl.program_id(0)
    j = pl.program_id(1)

    @pl.when((i == 0) & (j == 0))
    def _init():
        out_ref[...] = jnp.zeros((1, 1), jnp.float32)

    @pl.when(i <= j)
    def _compute():
        a = a_ref[...]
        bt = bt_ref[...]
        g2 = lax.dot_general(a, -2.0 * bt, (((1,), (0,)), ((), ())),
                             preferred_element_type=jnp.float32)
        sqa = jnp.sum(a * a, axis=1, keepdims=True)
        sqbe = jnp.sum(bt * bt, axis=0, keepdims=True) + D2_EPS
        d2 = jnp.maximum(sqa + (sqbe + g2), D2_EPS)
        r = SIGMA2 / d2
        inv6 = r * r * r
        e = inv6 * inv6 - inv6

        def diag_sum():
            rows = lax.broadcasted_iota(jnp.int32, (TI, TJ), 0)
            cols = lax.broadcasted_iota(jnp.int32, (TI, TJ), 1)
            return 0.5 * jnp.sum(jnp.where(rows == cols, 0.0, e))

        part = lax.cond(i == j, diag_sum, lambda: jnp.sum(e))
        out_ref[...] += (4.0 * EPSILON) * jnp.reshape(part, (1, 1))


def _energy_call(full_p, full_t):
    n_rows = full_p.shape[0]
    nbi = n_rows // TI
    nbj = n_rows // TJ
    out = pl.pallas_call(
        _energy_kernel,
        grid=(nbi, nbj),
        in_specs=[
            pl.BlockSpec((TI, PAD), lambda i, j: (i, 0)),
            pl.BlockSpec((PAD, TJ), lambda i, j: (0, j)),
        ],
        out_specs=pl.BlockSpec((1, 1), lambda i, j: (0, 0)),
        out_shape=jax.ShapeDtypeStruct((1, 1), jnp.float32),
    )(full_p, full_t)
    return out[0, 0]


def kernel(fg, bg, mask_idx):
    m_rows = fg.shape[0]
    n_rows = bg.shape[0]
    fgt = fg.astype(jnp.float32).T.reshape(-1)
    bg_p = jnp.pad(bg.astype(jnp.float32),
                   ((0, 0), (0, PAD - bg.shape[1]))).reshape(-1)
    idx = mask_idx.astype(jnp.int32)
    full_p = _make_scatter_sc(n_rows, m_rows)(fgt, bg_p, idx)
    full_p = full_p.reshape(n_rows, PAD)
    return _energy_call(full_p, full_p.T)


# speedup vs baseline: 2.5245x; 1.2507x over previous
import functools

import jax
import jax.numpy as jnp
from jax import lax
from jax.experimental import pallas as pl
from jax.experimental.pallas import tpu as pltpu
from jax.experimental.pallas import tpu_sc as plsc

SIGMA2 = 1.0
EPSILON = 0.25
D2_EPS = 1e-2

K = 8
LANES = 16
NW = 32


def _make_prep_sc(n_rows, m_rows):
    rpw = n_rows // NW
    mesh = plsc.VectorSubcoreMesh(core_axis_name="c", subcore_axis_name="s")

    @functools.partial(
        pl.kernel,
        out_type=jax.ShapeDtypeStruct((K * n_rows,), jnp.float32),
        mesh=mesh,
        scratch_types=[
            pltpu.VMEM((rpw * 3,), jnp.float32),
            pltpu.VMEM((m_rows,), jnp.int32),
            pltpu.VMEM((3 * m_rows,), jnp.float32),
            pltpu.VMEM((K * rpw,), jnp.float32),
        ],
        compiler_params=pltpu.CompilerParams(needs_layout_passes=False),
    )
    def prep(fgt_hbm, bgf_hbm, idx_hbm, c_hbm, bgl_v, idx_v, fgt_v, c_v):
        cid = lax.axis_index("c")
        sid = lax.axis_index("s")
        wid = sid * 2 + cid
        base = wid * rpw

        pltpu.sync_copy(bgf_hbm.at[pl.ds(base * 3, rpw * 3)], bgl_v)
        pltpu.sync_copy(idx_hbm, idx_v)
        pltpu.sync_copy(fgt_hbm, fgt_v)

        zeros_v = jnp.zeros((LANES,), jnp.float32)
        lane = lax.iota(jnp.int32, LANES)

        def fill(g, carry):
            xs = plsc.load_gather(bgl_v, [(g * LANES + lane) * 3])
            ys = plsc.load_gather(bgl_v, [(g * LANES + lane) * 3 + 1])
            zs = plsc.load_gather(bgl_v, [(g * LANES + lane) * 3 + 2])
            o = g * LANES
            c_v[pl.ds(0 * rpw + o, LANES)] = xs
            c_v[pl.ds(1 * rpw + o, LANES)] = ys
            c_v[pl.ds(2 * rpw + o, LANES)] = zs
            for c in range(3, K):
                c_v[pl.ds(c * rpw + o, LANES)] = zeros_v
            return carry

        lax.fori_loop(0, rpw // LANES, fill, 0)

        def merge(k, carry):
            tgt = idx_v[pl.ds(k * LANES, LANES)]
            ok = (tgt >= base) & (tgt < base + rpw)
            local = jnp.where(ok, tgt - base, 0)
            xs = fgt_v[pl.ds(0 * m_rows + k * LANES, LANES)]
            ys = fgt_v[pl.ds(1 * m_rows + k * LANES, LANES)]
            zs = fgt_v[pl.ds(2 * m_rows + k * LANES, LANES)]
            plsc.store_scatter(c_v, [0 * rpw + local], xs, mask=ok)
            plsc.store_scatter(c_v, [1 * rpw + local], ys, mask=ok)
            plsc.store_scatter(c_v, [2 * rpw + local], zs, mask=ok)
            return carry

        lax.fori_loop(0, m_rows // LANES, merge, 0)

        for c in range(K):
            pltpu.sync_copy(c_v.at[pl.ds(c * rpw, rpw)],
                            c_hbm.at[pl.ds(c * n_rows + base, rpw)])

    return prep


TI = 1024
TJ = 1024


def _energy_kernel(a_ref, b_ref, out_ref):
    i = pl.program_id(0)
    j = pl.program_id(1)

    @pl.when((i == 0) & (j == 0))
    def _init():
        out_ref[...] = jnp.zeros((1, 1), jnp.float32)

    @pl.when(i <= j)
    def _compute():
        a = a_ref[...]
        bt = b_ref[...]
        g2 = lax.dot_general(a, -2.0 * bt, (((1,), (0,)), ((), ())),
                             preferred_element_type=jnp.float32)
        sqa = jnp.sum(a * a, axis=1, keepdims=True)
        sqbe = jnp.sum(bt * bt, axis=0, keepdims=True) + D2_EPS
        d2 = jnp.maximum(sqa + (sqbe + g2), D2_EPS)
        r = SIGMA2 / d2
        r3 = r * r * r
        e = r3 * r3 - r3

        def diag_sum():
            rows = lax.broadcasted_iota(jnp.int32, (TI, TJ), 0)
            cols = lax.broadcasted_iota(jnp.int32, (TI, TJ), 1)
            return 0.5 * jnp.sum(jnp.where(rows == cols, 0.0, e))

        part = lax.cond(i == j, diag_sum, lambda: jnp.sum(e))
        out_ref[...] += (4.0 * EPSILON) * jnp.reshape(part, (1, 1))


def _energy_call(a_mat, b_mat):
    n_rows = a_mat.shape[0]
    nbi = n_rows // TI
    nbj = n_rows // TJ
    out = pl.pallas_call(
        _energy_kernel,
        grid=(nbi, nbj),
        in_specs=[
            pl.BlockSpec((TI, K), lambda i, j: (i, 0)),
            pl.BlockSpec((K, TJ), lambda i, j: (0, j)),
        ],
        out_specs=pl.BlockSpec((1, 1), lambda i, j: (0, 0)),
        out_shape=jax.ShapeDtypeStruct((1, 1), jnp.float32),
    )(a_mat, b_mat)
    return out[0, 0]


def kernel(fg, bg, mask_idx):
    m_rows = fg.shape[0]
    n_rows = bg.shape[0]
    fgt = fg.astype(jnp.float32).T.reshape(-1)
    bgf = bg.astype(jnp.float32).reshape(-1)
    idx = mask_idx.astype(jnp.int32)
    c_flat = _make_prep_sc(n_rows, m_rows)(fgt, bgf, idx)
    c_cm = c_flat.reshape(K, n_rows)
    return _energy_call(c_cm.T, c_cm)
